# Initial kernel scaffold; baseline (speedup 1.0000x reference)
import jax
import jax.numpy as jnp
from jax.experimental import pallas as pl


def kernel(x, pos_edge_index, neg_edge_index, x_param, W1_pl, W1_pr, b1_p, W1_nl, W1_nr, b1_n, W2_pl, W2_pr, b2_p, W2_nl, W2_nr, b2_n):
    raise NotImplementedError("write your pallas kernel here")


# speedup vs baseline: 17.1386x; 17.1386x over previous
import functools

import jax
import jax.numpy as jnp
from jax import lax
from jax.experimental import pallas as pl
from jax.experimental.pallas import tpu as pltpu
from jax.experimental.pallas import tpu_sc as plsc

N = 16896
IN_CH = 128
H2 = 32
E_POS = 540672
E_NEG = 270336

NC = 2
NS = 16
NW = NC * NS
CHUNK = 128

POS_ROWS = E_POS // CHUNK
NEG_ROWS = E_NEG // CHUNK
POS_PW = POS_ROWS // NW
NEG_PW = NEG_ROWS // NW
POS_PW_PAD = 136
NEG_PW_PAD = 72
STRIPE = N // NS


def _seg_sum_kernel(feat_w, with_deg):
    out_type = [
        jax.ShapeDtypeStruct((NC * N, feat_w), jnp.float32),
        jax.ShapeDtypeStruct((NC * N, feat_w), jnp.float32),
    ]
    scratch = [
        pltpu.VMEM((POS_PW_PAD, CHUNK), jnp.int32),
        pltpu.VMEM((POS_PW_PAD, CHUNK), jnp.int32),
        pltpu.VMEM((CHUNK, feat_w), jnp.float32),
        pltpu.VMEM_SHARED((N, feat_w), jnp.float32),
        pltpu.SemaphoreType.DMA,
    ]
    if with_deg:
        out_type += [
            jax.ShapeDtypeStruct((NC * N, 16), jnp.float32),
            jax.ShapeDtypeStruct((NC * N, 16), jnp.float32),
        ]
        scratch += [
            pltpu.VMEM((CHUNK, 16), jnp.float32),
            pltpu.VMEM_SHARED((N, 16), jnp.float32),
            pltpu.VMEM_SHARED((N, 16), jnp.float32),
        ]

    mesh = plsc.VectorSubcoreMesh(core_axis_name="c", subcore_axis_name="s",
                                  num_cores=NC, num_subcores=NS)

    def body(t_pos, t_neg, ps, pd, ns, nd, zf, zd, ones_h,
             out_p, out_n, *rest):
        if with_deg:
            deg_p_out, deg_n_out = rest[0], rest[1]
            rest = rest[2:]
        src_v, dst_v, rows_v, acc, sem = rest[:5]
        if with_deg:
            ones_v, deg_p, deg_n = rest[5:]

        cid = lax.axis_index("c")
        sid = lax.axis_index("s")
        wid = sid * NC + cid
        stripe = sid * STRIPE

        pltpu.sync_copy(zf, acc.at[pl.ds(stripe, STRIPE)])
        if with_deg:
            pltpu.sync_copy(zd, deg_p.at[pl.ds(stripe, STRIPE)])
            pltpu.sync_copy(zd, deg_n.at[pl.ds(stripe, STRIPE)])
            pltpu.sync_copy(ones_h, ones_v)
        plsc.subcore_barrier()

        def run_pass(table, src3d, dst3d, n_chunks, n_pad, deg_acc):
            pltpu.sync_copy(src3d.at[wid], src_v.at[pl.ds(0, n_pad)])
            pltpu.sync_copy(dst3d.at[wid], dst_v.at[pl.ds(0, n_pad)])

            def step(j, carry):
                pltpu.async_copy(table.at[src_v.at[j]], rows_v, sem).wait()
                pltpu.sync_copy(rows_v, acc.at[dst_v.at[j]], add=True)
                if deg_acc is not None:
                    pltpu.sync_copy(ones_v, deg_acc.at[dst_v.at[j]], add=True)
                return carry

            lax.fori_loop(0, n_chunks, step, 0)

        run_pass(t_pos, ps, pd, POS_PW, POS_PW_PAD,
                 deg_p if with_deg else None)
        plsc.subcore_barrier()
        pltpu.sync_copy(acc.at[pl.ds(stripe, STRIPE)],
                        out_p.at[pl.ds(cid * N + stripe, STRIPE)])
        if with_deg:
            pltpu.sync_copy(deg_p.at[pl.ds(stripe, STRIPE)],
                            deg_p_out.at[pl.ds(cid * N + stripe, STRIPE)])
        pltpu.sync_copy(zf, acc.at[pl.ds(stripe, STRIPE)])
        plsc.subcore_barrier()
        run_pass(t_neg, ns, nd, NEG_PW, NEG_PW_PAD,
                 deg_n if with_deg else None)
        plsc.subcore_barrier()
        pltpu.sync_copy(acc.at[pl.ds(stripe, STRIPE)],
                        out_n.at[pl.ds(cid * N + stripe, STRIPE)])
        if with_deg:
            pltpu.sync_copy(deg_n.at[pl.ds(stripe, STRIPE)],
                            deg_n_out.at[pl.ds(cid * N + stripe, STRIPE)])

    return pl.kernel(body, out_type=out_type, mesh=mesh,
                     scratch_types=scratch,
                     compiler_params=pltpu.CompilerParams(
                         use_tc_tiling_on_sc=False))


_seg32 = _seg_sum_kernel(H2, with_deg=True)
_seg64 = _seg_sum_kernel(2 * H2, with_deg=False)

ROW_BLK = 1056
GRID = N // ROW_BLK


def _proj_body(x_ref, wp_ref, wn_ref, tp_ref, tn_ref):
    x = x_ref[...]
    tp_ref[...] = jnp.dot(x, wp_ref[...], preferred_element_type=jnp.float32)
    tn_ref[...] = jnp.dot(x, wn_ref[...], preferred_element_type=jnp.float32)


_proj = pl.pallas_call(
    _proj_body,
    grid=(GRID,),
    in_specs=[
        pl.BlockSpec((ROW_BLK, IN_CH), lambda i: (i, 0)),
        pl.BlockSpec((IN_CH, H2), lambda i: (0, 0)),
        pl.BlockSpec((IN_CH, H2), lambda i: (0, 0)),
    ],
    out_specs=[
        pl.BlockSpec((ROW_BLK, H2), lambda i: (i, 0)),
        pl.BlockSpec((ROW_BLK, H2), lambda i: (i, 0)),
    ],
    out_shape=[
        jax.ShapeDtypeStruct((N, H2), jnp.float32),
        jax.ShapeDtypeStruct((N, H2), jnp.float32),
    ],
)


def _recip_deg(deg_blk):
    d = deg_blk[0, :, 0:1] + deg_blk[1, :, 0:1]
    return 1.0 / jnp.maximum(d, 1.0)


def _z_body(ap_ref, an_ref, dp_ref, dn_ref, x_ref, wr_ref, b_ref, z_ref):
    sp = ap_ref[0] + ap_ref[1]
    sn = an_ref[0] + an_ref[1]
    rp = _recip_deg(dp_ref[...])
    rn = _recip_deg(dn_ref[...])
    agg = jnp.concatenate([sp * rp, sn * rn], axis=1)
    lin = jnp.dot(x_ref[...], wr_ref[...], preferred_element_type=jnp.float32)
    z_ref[...] = jnp.maximum(agg + lin + b_ref[...], 0.0)


_zker = pl.pallas_call(
    _z_body,
    grid=(GRID,),
    in_specs=[
        pl.BlockSpec((2, ROW_BLK, H2), lambda i: (0, i, 0)),
        pl.BlockSpec((2, ROW_BLK, H2), lambda i: (0, i, 0)),
        pl.BlockSpec((2, ROW_BLK, 16), lambda i: (0, i, 0)),
        pl.BlockSpec((2, ROW_BLK, 16), lambda i: (0, i, 0)),
        pl.BlockSpec((ROW_BLK, IN_CH), lambda i: (i, 0)),
        pl.BlockSpec((IN_CH, 2 * H2), lambda i: (0, 0)),
        pl.BlockSpec((1, 2 * H2), lambda i: (0, 0)),
    ],
    out_specs=pl.BlockSpec((ROW_BLK, 2 * H2), lambda i: (i, 0)),
    out_shape=jax.ShapeDtypeStruct((N, 2 * H2), jnp.float32),
)


def _final_body(ap_ref, an_ref, dp_ref, dn_ref, z_ref, m1_ref, m2_ref,
                m3_ref, b_ref, out_ref):
    rp = _recip_deg(dp_ref[...])
    rn = _recip_deg(dn_ref[...])
    mp = (ap_ref[0] + ap_ref[1]) * rp
    mn = (an_ref[0] + an_ref[1]) * rn
    acc = jnp.dot(mp, m1_ref[...], preferred_element_type=jnp.float32)
    acc += jnp.dot(mn, m2_ref[...], preferred_element_type=jnp.float32)
    acc += jnp.dot(z_ref[...], m3_ref[...], preferred_element_type=jnp.float32)
    out_ref[...] = jnp.maximum(acc + b_ref[...], 0.0)


_finalker = pl.pallas_call(
    _final_body,
    grid=(GRID,),
    in_specs=[
        pl.BlockSpec((2, ROW_BLK, 2 * H2), lambda i: (0, i, 0)),
        pl.BlockSpec((2, ROW_BLK, 2 * H2), lambda i: (0, i, 0)),
        pl.BlockSpec((2, ROW_BLK, 16), lambda i: (0, i, 0)),
        pl.BlockSpec((2, ROW_BLK, 16), lambda i: (0, i, 0)),
        pl.BlockSpec((ROW_BLK, 2 * H2), lambda i: (i, 0)),
        pl.BlockSpec((2 * H2, 2 * H2), lambda i: (0, 0)),
        pl.BlockSpec((2 * H2, 2 * H2), lambda i: (0, 0)),
        pl.BlockSpec((2 * H2, 2 * H2), lambda i: (0, 0)),
        pl.BlockSpec((1, 2 * H2), lambda i: (0, 0)),
    ],
    out_specs=pl.BlockSpec((ROW_BLK, 2 * H2), lambda i: (i, 0)),
    out_shape=jax.ShapeDtypeStruct((N, 2 * H2), jnp.float32),
)


def kernel(x, pos_edge_index, neg_edge_index, x_param, W1_pl, W1_pr, b1_p,
           W1_nl, W1_nr, b1_n, W2_pl, W2_pr, b2_p, W2_nl, W2_nr, b2_n):
    del x

    def _split(idx_row, pw, pw_pad):
        a = idx_row.reshape(NW, pw, CHUNK)
        return jnp.pad(a, ((0, 0), (0, pw_pad - pw), (0, 0)))

    ps = _split(pos_edge_index[0], POS_PW, POS_PW_PAD)
    pd = _split(pos_edge_index[1], POS_PW, POS_PW_PAD)
    ns = _split(neg_edge_index[0], NEG_PW, NEG_PW_PAD)
    nd = _split(neg_edge_index[1], NEG_PW, NEG_PW_PAD)

    zf32 = jnp.zeros((STRIPE, H2), jnp.float32)
    zf64 = jnp.zeros((STRIPE, 2 * H2), jnp.float32)
    zd = jnp.zeros((STRIPE, 16), jnp.float32)
    ones_h = jnp.ones((CHUNK, 16), jnp.float32)

    t_pos, t_neg = _proj(x_param, W1_pl, W1_nl)
    accp, accn, degp, degn = _seg32(t_pos, t_neg, ps, pd, ns, nd,
                                    zf32, zd, ones_h)
    accp = accp.reshape(NC, N, H2)
    accn = accn.reshape(NC, N, H2)
    degp = degp.reshape(NC, N, 16)
    degn = degn.reshape(NC, N, 16)

    wr1 = jnp.concatenate([W1_pr, W1_nr], axis=1)
    b1 = jnp.concatenate([b1_p, b1_n]).reshape(1, 2 * H2)
    z = _zker(accp, accn, degp, degn, x_param, wr1, b1)

    accp2, accn2 = _seg64(z, z, ps, pd, ns, nd, zf64, zd, ones_h)
    accp2 = accp2.reshape(NC, N, 2 * H2)
    accn2 = accn2.reshape(NC, N, 2 * H2)

    zero = jnp.zeros((H2, H2), jnp.float32)
    A, B = W2_pl[:H2], W2_pl[H2:]
    C, D = W2_nl[:H2], W2_nl[H2:]
    m1 = jnp.concatenate([jnp.concatenate([A, zero], axis=1),
                          jnp.concatenate([zero, C], axis=1)], axis=0)
    m2 = jnp.concatenate([jnp.concatenate([zero, D], axis=1),
                          jnp.concatenate([B, zero], axis=1)], axis=0)
    m3 = jnp.concatenate([jnp.concatenate([W2_pr, zero], axis=1),
                          jnp.concatenate([zero, W2_nr], axis=1)], axis=0)
    b2 = jnp.concatenate([b2_p, b2_n]).reshape(1, 2 * H2)

    return _finalker(accp2, accn2, degp, degn, z, m1, m2, m3, b2)


# speedup vs baseline: 24.3499x; 1.4208x over previous
import functools

import jax
import jax.numpy as jnp
from jax import lax
from jax.experimental import pallas as pl
from jax.experimental.pallas import tpu as pltpu
from jax.experimental.pallas import tpu_sc as plsc

N = 16896
IN_CH = 128
H2 = 32
E_POS = 540672
E_NEG = 270336

NC = 2
NS = 16
NW = NC * NS
CHUNK = 128

POS_ROWS = E_POS // CHUNK
NEG_ROWS = E_NEG // CHUNK
POS_PW = POS_ROWS // NW
NEG_PW = NEG_ROWS // NW
POS_PW_PAD = 136
NEG_PW_PAD = 72
STRIPE = N // NS


def _seg_sum_kernel(feat_w, with_deg):
    out_type = [
        jax.ShapeDtypeStruct((NC * N, feat_w), jnp.float32),
        jax.ShapeDtypeStruct((NC * N, feat_w), jnp.float32),
    ]
    n_acc = 1
    scratch = [
        pltpu.VMEM((POS_PW_PAD, CHUNK), jnp.int32),
        pltpu.VMEM((POS_PW_PAD, CHUNK), jnp.int32),
        pltpu.VMEM((CHUNK, feat_w), jnp.float32),
        pltpu.VMEM((CHUNK, feat_w), jnp.float32),
    ] + [pltpu.VMEM_SHARED((N, feat_w), jnp.float32)
         for _ in range(n_acc)] + [
        pltpu.SemaphoreType.DMA,
        pltpu.SemaphoreType.DMA,
        pltpu.SemaphoreType.DMA,
        pltpu.SemaphoreType.DMA,
    ]
    if with_deg:
        out_type += [
            jax.ShapeDtypeStruct((NC * N, 16), jnp.float32),
            jax.ShapeDtypeStruct((NC * N, 16), jnp.float32),
        ]
        scratch += [
            pltpu.VMEM((CHUNK, 16), jnp.float32),
            pltpu.VMEM_SHARED((N, 16), jnp.float32),
            pltpu.VMEM_SHARED((N, 16), jnp.float32),
            pltpu.SemaphoreType.DMA,
        ]

    mesh = plsc.VectorSubcoreMesh(core_axis_name="c", subcore_axis_name="s",
                                  num_cores=NC, num_subcores=NS)

    def body(t_pos, t_neg, ps, pd, ns, nd, zf, zd, ones_h,
             out_p, out_n, *rest):
        if with_deg:
            deg_p_out, deg_n_out = rest[0], rest[1]
            rest = rest[2:]
        src_v, dst_v, rows0, rows1 = rest[:4]
        rest = rest[4:]
        accs = rest[:n_acc]
        rest = rest[n_acc:]
        gs0, gs1, ss0, ss1 = rest[:4]
        if with_deg:
            ones_v, deg_p, deg_n, ds = rest[4:]

        cid = lax.axis_index("c")
        sid = lax.axis_index("s")
        wid = sid * NC + cid
        stripe = sid * STRIPE

        for a in accs:
            pltpu.sync_copy(zf, a.at[pl.ds(stripe, STRIPE)])
        if with_deg:
            pltpu.sync_copy(zd, deg_p.at[pl.ds(stripe, STRIPE)])
            pltpu.sync_copy(zd, deg_n.at[pl.ds(stripe, STRIPE)])
            pltpu.sync_copy(ones_h, ones_v)
        plsc.subcore_barrier()

        def run_pass(table, src3d, dst3d, n_chunks, n_pad, acc, deg_acc):
            pltpu.sync_copy(src3d.at[wid], src_v.at[pl.ds(0, n_pad)])
            pltpu.sync_copy(dst3d.at[wid], dst_v.at[pl.ds(0, n_pad)])

            bufs = ((rows0, gs0, ss0), (rows1, gs1, ss1))

            def gather(c, buf, gsem):
                return pltpu.async_copy(table.at[src_v.at[c]], buf, gsem)

            gather(0, rows0, gs0)
            gather(1, rows1, gs1)

            n2 = n_chunks // 2

            def iter_g(g, carry):
                for k, (buf, gsem, ssem) in enumerate(bufs):
                    c = 2 * g + k
                    pltpu.make_async_copy(table.at[src_v.at[c]],
                                          buf, gsem).wait()
                    pltpu.sync_copy(buf, acc.at[dst_v.at[c]], add=True)
                    if deg_acc is not None:
                        pltpu.sync_copy(ones_v, deg_acc.at[dst_v.at[c]],
                                        add=True)

                    @pl.when(g < n2 - 1)
                    def _():
                        gather(c + 2, buf, gsem)
                return carry

            lax.fori_loop(0, n2, iter_g, 0)

        def write_out(sh_acc, hbm_out):
            pltpu.sync_copy(sh_acc.at[pl.ds(stripe, STRIPE)],
                            hbm_out.at[pl.ds(cid * N + stripe, STRIPE)])

        run_pass(t_pos, ps, pd, POS_PW, POS_PW_PAD, accs[0],
                 deg_p if with_deg else None)
        plsc.subcore_barrier()
        write_out(accs[0], out_p)
        if with_deg:
            write_out(deg_p, deg_p_out)
        pltpu.sync_copy(zf, accs[0].at[pl.ds(stripe, STRIPE)])
        plsc.subcore_barrier()
        run_pass(t_neg, ns, nd, NEG_PW, NEG_PW_PAD, accs[0],
                 deg_n if with_deg else None)
        plsc.subcore_barrier()
        write_out(accs[0], out_n)
        if with_deg:
            write_out(deg_n, deg_n_out)

    return pl.kernel(body, out_type=out_type, mesh=mesh,
                     scratch_types=scratch,
                     compiler_params=pltpu.CompilerParams(
                         use_tc_tiling_on_sc=False))


_seg32 = _seg_sum_kernel(H2, with_deg=True)
_seg64 = _seg_sum_kernel(2 * H2, with_deg=False)

ROW_BLK = 1056
GRID = N // ROW_BLK


def _proj_body(x_ref, wp_ref, wn_ref, tp_ref, tn_ref):
    x = x_ref[...]
    tp_ref[...] = jnp.dot(x, wp_ref[...], preferred_element_type=jnp.float32)
    tn_ref[...] = jnp.dot(x, wn_ref[...], preferred_element_type=jnp.float32)


_proj = pl.pallas_call(
    _proj_body,
    grid=(GRID,),
    in_specs=[
        pl.BlockSpec((ROW_BLK, IN_CH), lambda i: (i, 0)),
        pl.BlockSpec((IN_CH, H2), lambda i: (0, 0)),
        pl.BlockSpec((IN_CH, H2), lambda i: (0, 0)),
    ],
    out_specs=[
        pl.BlockSpec((ROW_BLK, H2), lambda i: (i, 0)),
        pl.BlockSpec((ROW_BLK, H2), lambda i: (i, 0)),
    ],
    out_shape=[
        jax.ShapeDtypeStruct((N, H2), jnp.float32),
        jax.ShapeDtypeStruct((N, H2), jnp.float32),
    ],
)


def _recip_deg(deg_blk):
    d = deg_blk[0, :, 0:1] + deg_blk[1, :, 0:1]
    return 1.0 / jnp.maximum(d, 1.0)


def _z_body(ap_ref, an_ref, dp_ref, dn_ref, x_ref, wr_ref, b_ref, z_ref):
    sp = ap_ref[0] + ap_ref[1]
    sn = an_ref[0] + an_ref[1]
    rp = _recip_deg(dp_ref[...])
    rn = _recip_deg(dn_ref[...])
    agg = jnp.concatenate([sp * rp, sn * rn], axis=1)
    lin = jnp.dot(x_ref[...], wr_ref[...], preferred_element_type=jnp.float32)
    z_ref[...] = jnp.maximum(agg + lin + b_ref[...], 0.0)


_zker = pl.pallas_call(
    _z_body,
    grid=(GRID,),
    in_specs=[
        pl.BlockSpec((2, ROW_BLK, H2), lambda i: (0, i, 0)),
        pl.BlockSpec((2, ROW_BLK, H2), lambda i: (0, i, 0)),
        pl.BlockSpec((2, ROW_BLK, 16), lambda i: (0, i, 0)),
        pl.BlockSpec((2, ROW_BLK, 16), lambda i: (0, i, 0)),
        pl.BlockSpec((ROW_BLK, IN_CH), lambda i: (i, 0)),
        pl.BlockSpec((IN_CH, 2 * H2), lambda i: (0, 0)),
        pl.BlockSpec((1, 2 * H2), lambda i: (0, 0)),
    ],
    out_specs=pl.BlockSpec((ROW_BLK, 2 * H2), lambda i: (i, 0)),
    out_shape=jax.ShapeDtypeStruct((N, 2 * H2), jnp.float32),
)


def _final_body(ap_ref, an_ref, dp_ref, dn_ref, z_ref, m1_ref, m2_ref,
                m3_ref, b_ref, out_ref):
    rp = _recip_deg(dp_ref[...])
    rn = _recip_deg(dn_ref[...])
    mp = (ap_ref[0] + ap_ref[1]) * rp
    mn = (an_ref[0] + an_ref[1]) * rn
    acc = jnp.dot(mp, m1_ref[...], preferred_element_type=jnp.float32)
    acc += jnp.dot(mn, m2_ref[...], preferred_element_type=jnp.float32)
    acc += jnp.dot(z_ref[...], m3_ref[...], preferred_element_type=jnp.float32)
    out_ref[...] = jnp.maximum(acc + b_ref[...], 0.0)


_finalker = pl.pallas_call(
    _final_body,
    grid=(GRID,),
    in_specs=[
        pl.BlockSpec((2, ROW_BLK, 2 * H2), lambda i: (0, i, 0)),
        pl.BlockSpec((2, ROW_BLK, 2 * H2), lambda i: (0, i, 0)),
        pl.BlockSpec((2, ROW_BLK, 16), lambda i: (0, i, 0)),
        pl.BlockSpec((2, ROW_BLK, 16), lambda i: (0, i, 0)),
        pl.BlockSpec((ROW_BLK, 2 * H2), lambda i: (i, 0)),
        pl.BlockSpec((2 * H2, 2 * H2), lambda i: (0, 0)),
        pl.BlockSpec((2 * H2, 2 * H2), lambda i: (0, 0)),
        pl.BlockSpec((2 * H2, 2 * H2), lambda i: (0, 0)),
        pl.BlockSpec((1, 2 * H2), lambda i: (0, 0)),
    ],
    out_specs=pl.BlockSpec((ROW_BLK, 2 * H2), lambda i: (i, 0)),
    out_shape=jax.ShapeDtypeStruct((N, 2 * H2), jnp.float32),
)


def kernel(x, pos_edge_index, neg_edge_index, x_param, W1_pl, W1_pr, b1_p,
           W1_nl, W1_nr, b1_n, W2_pl, W2_pr, b2_p, W2_nl, W2_nr, b2_n):
    del x

    def _split(idx_row, pw, pw_pad):
        a = idx_row.reshape(NW, pw, CHUNK)
        return jnp.pad(a, ((0, 0), (0, pw_pad - pw), (0, 0)))

    ps = _split(pos_edge_index[0], POS_PW, POS_PW_PAD)
    pd = _split(pos_edge_index[1], POS_PW, POS_PW_PAD)
    ns = _split(neg_edge_index[0], NEG_PW, NEG_PW_PAD)
    nd = _split(neg_edge_index[1], NEG_PW, NEG_PW_PAD)

    zf32 = jnp.zeros((STRIPE, H2), jnp.float32)
    zf64 = jnp.zeros((STRIPE, 2 * H2), jnp.float32)
    zd = jnp.zeros((STRIPE, 16), jnp.float32)
    ones_h = jnp.ones((CHUNK, 16), jnp.float32)

    t_pos, t_neg = _proj(x_param, W1_pl, W1_nl)
    accp, accn, degp, degn = _seg32(t_pos, t_neg, ps, pd, ns, nd,
                                    zf32, zd, ones_h)
    accp = accp.reshape(NC, N, H2)
    accn = accn.reshape(NC, N, H2)
    degp = degp.reshape(NC, N, 16)
    degn = degn.reshape(NC, N, 16)

    wr1 = jnp.concatenate([W1_pr, W1_nr], axis=1)
    b1 = jnp.concatenate([b1_p, b1_n]).reshape(1, 2 * H2)
    z = _zker(accp, accn, degp, degn, x_param, wr1, b1)

    accp2, accn2 = _seg64(z, z, ps, pd, ns, nd, zf64, zd, ones_h)
    accp2 = accp2.reshape(NC, N, 2 * H2)
    accn2 = accn2.reshape(NC, N, 2 * H2)

    zero = jnp.zeros((H2, H2), jnp.float32)
    A, B = W2_pl[:H2], W2_pl[H2:]
    C, D = W2_nl[:H2], W2_nl[H2:]
    m1 = jnp.concatenate([jnp.concatenate([A, zero], axis=1),
                          jnp.concatenate([zero, C], axis=1)], axis=0)
    m2 = jnp.concatenate([jnp.concatenate([zero, D], axis=1),
                          jnp.concatenate([B, zero], axis=1)], axis=0)
    m3 = jnp.concatenate([jnp.concatenate([W2_pr, zero], axis=1),
                          jnp.concatenate([zero, W2_nr], axis=1)], axis=0)
    b2 = jnp.concatenate([b2_p, b2_n]).reshape(1, 2 * H2)

    return _finalker(accp2, accn2, degp, degn, z, m1, m2, m3, b2)


# speedup vs baseline: 25.4148x; 1.0437x over previous
import functools

import jax
import jax.numpy as jnp
from jax import lax
from jax.experimental import pallas as pl
from jax.experimental.pallas import tpu as pltpu
from jax.experimental.pallas import tpu_sc as plsc

N = 16896
IN_CH = 128
H2 = 32
E_POS = 540672
E_NEG = 270336

NC = 2
NS = 16
NW = NC * NS
CHUNK = 128

POS_ROWS = E_POS // CHUNK
NEG_ROWS = E_NEG // CHUNK
POS_PW = POS_ROWS // NW
NEG_PW = NEG_ROWS // NW
POS_STAGE = POS_PW + 4
NEG_STAGE = NEG_PW + 6
STRIPE = N // NS


def _seg_sum_kernel(feat_w, with_deg):
    out_type = [
        jax.ShapeDtypeStruct((NC * N, feat_w), jnp.float32),
        jax.ShapeDtypeStruct((NC * N, feat_w), jnp.float32),
    ]
    scratch = [
        pltpu.VMEM((POS_STAGE, CHUNK), jnp.int32),
        pltpu.VMEM((POS_STAGE, CHUNK), jnp.int32),
        pltpu.VMEM((CHUNK, feat_w), jnp.float32),
        pltpu.VMEM((CHUNK, feat_w), jnp.float32),
        pltpu.VMEM_SHARED((N, feat_w), jnp.float32),
        pltpu.SemaphoreType.DMA,
        pltpu.SemaphoreType.DMA,
    ]
    if with_deg:
        out_type += [
            jax.ShapeDtypeStruct((NC * N, 16), jnp.float32),
            jax.ShapeDtypeStruct((NC * N, 16), jnp.float32),
        ]
        scratch += [
            pltpu.VMEM((CHUNK, 16), jnp.float32),
            pltpu.VMEM_SHARED((N, 16), jnp.float32),
            pltpu.VMEM_SHARED((N, 16), jnp.float32),
        ]

    mesh = plsc.VectorSubcoreMesh(core_axis_name="c", subcore_axis_name="s",
                                  num_cores=NC, num_subcores=NS)

    def body(t_pos, t_neg, pos_idx, neg_idx, zf, zd, ones_h,
             out_p, out_n, *rest):
        if with_deg:
            deg_p_out, deg_n_out = rest[0], rest[1]
            rest = rest[2:]
        src_v, dst_v, rows0, rows1, acc, gs0, gs1 = rest[:7]
        if with_deg:
            ones_v, deg_p, deg_n = rest[7:]

        cid = lax.axis_index("c")
        sid = lax.axis_index("s")
        wid = sid * NC + cid
        stripe = sid * STRIPE

        pltpu.sync_copy(zf, acc.at[pl.ds(stripe, STRIPE)])
        if with_deg:
            pltpu.sync_copy(zd, deg_p.at[pl.ds(stripe, STRIPE)])
            pltpu.sync_copy(zd, deg_n.at[pl.ds(stripe, STRIPE)])
            pltpu.sync_copy(ones_h, ones_v)
        plsc.subcore_barrier()

        def run_pass(table, idx2d, pw, n_rows, stage, deg_acc):
            base = (pw * wid) // 8 * 8
            n_w = (pw * wid + pw) // 8 * 8 - base
            pltpu.sync_copy(idx2d.at[pl.ds(base, stage)],
                            src_v.at[pl.ds(0, stage)])
            pltpu.sync_copy(idx2d.at[pl.ds(n_rows + base, stage)],
                            dst_v.at[pl.ds(0, stage)])

            bufs = ((rows0, gs0), (rows1, gs1))

            def gather(c, buf, gsem):
                return pltpu.async_copy(table.at[src_v.at[c]], buf, gsem)

            gather(0, rows0, gs0)
            gather(1, rows1, gs1)

            n2 = n_w // 2

            def iter_g(g, carry):
                for k, (buf, gsem) in enumerate(bufs):
                    c = 2 * g + k
                    pltpu.make_async_copy(table.at[src_v.at[c]],
                                          buf, gsem).wait()
                    pltpu.sync_copy(buf, acc.at[dst_v.at[c]], add=True)
                    if deg_acc is not None:
                        pltpu.sync_copy(ones_v, deg_acc.at[dst_v.at[c]],
                                        add=True)

                    @pl.when(g < n2 - 1)
                    def _():
                        gather(c + 2, buf, gsem)
                return carry

            lax.fori_loop(0, n2, iter_g, 0)

        def write_out(sh_acc, hbm_out):
            pltpu.sync_copy(sh_acc.at[pl.ds(stripe, STRIPE)],
                            hbm_out.at[pl.ds(cid * N + stripe, STRIPE)])

        run_pass(t_pos, pos_idx, POS_PW, POS_ROWS, POS_STAGE,
                 deg_p if with_deg else None)
        plsc.subcore_barrier()
        write_out(acc, out_p)
        if with_deg:
            write_out(deg_p, deg_p_out)
        pltpu.sync_copy(zf, acc.at[pl.ds(stripe, STRIPE)])
        plsc.subcore_barrier()
        run_pass(t_neg, neg_idx, NEG_PW, NEG_ROWS, NEG_STAGE,
                 deg_n if with_deg else None)
        plsc.subcore_barrier()
        write_out(acc, out_n)
        if with_deg:
            write_out(deg_n, deg_n_out)

    return pl.kernel(body, out_type=out_type, mesh=mesh,
                     scratch_types=scratch,
                     compiler_params=pltpu.CompilerParams(
                         use_tc_tiling_on_sc=False))


_seg32 = _seg_sum_kernel(H2, with_deg=True)
_seg64 = _seg_sum_kernel(2 * H2, with_deg=False)

ROW_BLK = 1056
GRID = N // ROW_BLK
NBLK = N // ROW_BLK


def _proj_body(x_ref, wp_ref, wn_ref, wr_ref, b_ref,
               tp_ref, tn_ref, lin_ref):
    x = x_ref[...]
    tp_ref[...] = jnp.dot(x, wp_ref[...], preferred_element_type=jnp.float32)
    tn_ref[...] = jnp.dot(x, wn_ref[...], preferred_element_type=jnp.float32)
    lin_ref[...] = (jnp.dot(x, wr_ref[...], preferred_element_type=jnp.float32)
                    + b_ref[...])


_proj = pl.pallas_call(
    _proj_body,
    grid=(GRID,),
    in_specs=[
        pl.BlockSpec((ROW_BLK, IN_CH), lambda i: (i, 0)),
        pl.BlockSpec((IN_CH, H2), lambda i: (0, 0)),
        pl.BlockSpec((IN_CH, H2), lambda i: (0, 0)),
        pl.BlockSpec((IN_CH, 2 * H2), lambda i: (0, 0)),
        pl.BlockSpec((1, 2 * H2), lambda i: (0, 0)),
    ],
    out_specs=[
        pl.BlockSpec((ROW_BLK, H2), lambda i: (i, 0)),
        pl.BlockSpec((ROW_BLK, H2), lambda i: (i, 0)),
        pl.BlockSpec((ROW_BLK, 2 * H2), lambda i: (i, 0)),
    ],
    out_shape=[
        jax.ShapeDtypeStruct((N, H2), jnp.float32),
        jax.ShapeDtypeStruct((N, H2), jnp.float32),
        jax.ShapeDtypeStruct((N, 2 * H2), jnp.float32),
    ],
)


def _pair_specs(feat_w):
    return [
        pl.BlockSpec((ROW_BLK, feat_w), lambda i: (i, 0)),
        pl.BlockSpec((ROW_BLK, feat_w), lambda i: (NBLK + i, 0)),
    ]


def _recip_deg(d0, d1):
    d = d0[:, 0:1] + d1[:, 0:1]
    return 1.0 / jnp.maximum(d, 1.0)


def _z_body(ap0_ref, ap1_ref, an0_ref, an1_ref, dp0_ref, dp1_ref,
            dn0_ref, dn1_ref, lin_ref, z_ref):
    sp = ap0_ref[...] + ap1_ref[...]
    sn = an0_ref[...] + an1_ref[...]
    rp = _recip_deg(dp0_ref[...], dp1_ref[...])
    rn = _recip_deg(dn0_ref[...], dn1_ref[...])
    agg = jnp.concatenate([sp * rp, sn * rn], axis=1)
    z_ref[...] = jnp.maximum(agg + lin_ref[...], 0.0)


_zker = pl.pallas_call(
    _z_body,
    grid=(GRID,),
    in_specs=(_pair_specs(H2) + _pair_specs(H2)
              + _pair_specs(16) + _pair_specs(16)
              + [pl.BlockSpec((ROW_BLK, 2 * H2), lambda i: (i, 0))]),
    out_specs=pl.BlockSpec((ROW_BLK, 2 * H2), lambda i: (i, 0)),
    out_shape=jax.ShapeDtypeStruct((N, 2 * H2), jnp.float32),
)


def _final_body(ap0_ref, ap1_ref, an0_ref, an1_ref, dp0_ref, dp1_ref,
                dn0_ref, dn1_ref, z_ref, m1_ref, m2_ref, m3_ref, b_ref,
                out_ref):
    rp = _recip_deg(dp0_ref[...], dp1_ref[...])
    rn = _recip_deg(dn0_ref[...], dn1_ref[...])
    mp = (ap0_ref[...] + ap1_ref[...]) * rp
    mn = (an0_ref[...] + an1_ref[...]) * rn
    acc = jnp.dot(mp, m1_ref[...], preferred_element_type=jnp.float32)
    acc += jnp.dot(mn, m2_ref[...], preferred_element_type=jnp.float32)
    acc += jnp.dot(z_ref[...], m3_ref[...], preferred_element_type=jnp.float32)
    out_ref[...] = jnp.maximum(acc + b_ref[...], 0.0)


_finalker = pl.pallas_call(
    _final_body,
    grid=(GRID,),
    in_specs=(_pair_specs(2 * H2) + _pair_specs(2 * H2)
              + _pair_specs(16) + _pair_specs(16)
              + [
                  pl.BlockSpec((ROW_BLK, 2 * H2), lambda i: (i, 0)),
                  pl.BlockSpec((2 * H2, 2 * H2), lambda i: (0, 0)),
                  pl.BlockSpec((2 * H2, 2 * H2), lambda i: (0, 0)),
                  pl.BlockSpec((2 * H2, 2 * H2), lambda i: (0, 0)),
                  pl.BlockSpec((1, 2 * H2), lambda i: (0, 0)),
              ]),
    out_specs=pl.BlockSpec((ROW_BLK, 2 * H2), lambda i: (i, 0)),
    out_shape=jax.ShapeDtypeStruct((N, 2 * H2), jnp.float32),
)


def kernel(x, pos_edge_index, neg_edge_index, x_param, W1_pl, W1_pr, b1_p,
           W1_nl, W1_nr, b1_n, W2_pl, W2_pr, b2_p, W2_nl, W2_nr, b2_n):
    del x

    pos_idx = pos_edge_index.reshape(2 * POS_ROWS, CHUNK)
    neg_idx = neg_edge_index.reshape(2 * NEG_ROWS, CHUNK)

    zf32 = jnp.zeros((STRIPE, H2), jnp.float32)
    zf64 = jnp.zeros((STRIPE, 2 * H2), jnp.float32)
    zd = jnp.zeros((STRIPE, 16), jnp.float32)
    ones_h = jnp.ones((CHUNK, 16), jnp.float32)

    wr1 = jnp.concatenate([W1_pr, W1_nr], axis=1)
    b1 = jnp.concatenate([b1_p, b1_n]).reshape(1, 2 * H2)

    t_pos, t_neg, lin = _proj(x_param, W1_pl, W1_nl, wr1, b1)
    accp, accn, degp, degn = _seg32(t_pos, t_neg, pos_idx, neg_idx,
                                    zf32, zd, ones_h)

    z = _zker(accp, accp, accn, accn, degp, degp, degn, degn, lin)

    accp2, accn2 = _seg64(z, z, pos_idx, neg_idx, zf64, zd, ones_h)

    zero = jnp.zeros((H2, H2), jnp.float32)
    A, B = W2_pl[:H2], W2_pl[H2:]
    C, D = W2_nl[:H2], W2_nl[H2:]
    m1 = jnp.concatenate([jnp.concatenate([A, zero], axis=1),
                          jnp.concatenate([zero, C], axis=1)], axis=0)
    m2 = jnp.concatenate([jnp.concatenate([zero, D], axis=1),
                          jnp.concatenate([B, zero], axis=1)], axis=0)
    m3 = jnp.concatenate([jnp.concatenate([W2_pr, zero], axis=1),
                          jnp.concatenate([zero, W2_nr], axis=1)], axis=0)
    b2 = jnp.concatenate([b2_p, b2_n]).reshape(1, 2 * H2)

    return _finalker(accp2, accp2, accn2, accn2, degp, degp, degn, degn,
                     z, m1, m2, m3, b2)


# speedup vs baseline: 26.0409x; 1.0246x over previous
import functools

import jax
import jax.numpy as jnp
from jax import lax
from jax.experimental import pallas as pl
from jax.experimental.pallas import tpu as pltpu
from jax.experimental.pallas import tpu_sc as plsc

N = 16896
IN_CH = 128
H2 = 32
E_POS = 540672
E_NEG = 270336

NC = 2
NS = 16
NW = NC * NS
CHUNK = 128

POS_ROWS = E_POS // CHUNK
NEG_ROWS = E_NEG // CHUNK
POS_PW = POS_ROWS // NW
NEG_PW = NEG_ROWS // NW
POS_STAGE = POS_PW + 4
NEG_STAGE = NEG_PW + 6
STRIPE = N // NS


def _seg_sum_kernel(feat_w, with_deg):
    out_type = [
        jax.ShapeDtypeStruct((NC * N, feat_w), jnp.float32),
        jax.ShapeDtypeStruct((NC * N, feat_w), jnp.float32),
    ]
    scratch = [
        pltpu.VMEM((POS_STAGE, CHUNK), jnp.int32),
        pltpu.VMEM((POS_STAGE, CHUNK), jnp.int32),
        pltpu.VMEM((CHUNK, feat_w), jnp.float32),
        pltpu.VMEM((CHUNK, feat_w), jnp.float32),
        pltpu.VMEM_SHARED((N, feat_w), jnp.float32),
        pltpu.SemaphoreType.DMA,
        pltpu.SemaphoreType.DMA,
    ]
    if with_deg:
        out_type += [
            jax.ShapeDtypeStruct((NC * N, 16), jnp.float32),
            jax.ShapeDtypeStruct((NC * N, 16), jnp.float32),
        ]
        scratch += [
            pltpu.VMEM((CHUNK, 16), jnp.float32),
            pltpu.VMEM_SHARED((N, 16), jnp.float32),
            pltpu.VMEM_SHARED((N, 16), jnp.float32),
            pltpu.SemaphoreType.DMA,
        ]

    mesh = plsc.VectorSubcoreMesh(core_axis_name="c", subcore_axis_name="s",
                                  num_cores=NC, num_subcores=NS)

    def body(t_pos, t_neg, pos_idx, neg_idx, zf, zd, ones_h,
             out_p, out_n, *rest):
        if with_deg:
            deg_p_out, deg_n_out = rest[0], rest[1]
            rest = rest[2:]
        src_v, dst_v, rows0, rows1, acc, gs0, gs1 = rest[:7]
        if with_deg:
            ones_v, deg_p, deg_n, ds = rest[7:]

        cid = lax.axis_index("c")
        sid = lax.axis_index("s")
        wid = sid * NC + cid
        stripe = sid * STRIPE

        pltpu.sync_copy(zf, acc.at[pl.ds(stripe, STRIPE)])
        if with_deg:
            pltpu.sync_copy(zd, deg_p.at[pl.ds(stripe, STRIPE)])
            pltpu.sync_copy(zd, deg_n.at[pl.ds(stripe, STRIPE)])
            pltpu.sync_copy(ones_h, ones_v)
        plsc.subcore_barrier()

        def run_pass(table, idx2d, pw, n_rows, stage, deg_acc):
            base = (pw * wid) // 8 * 8
            n_w = (pw * wid + pw) // 8 * 8 - base
            pltpu.sync_copy(idx2d.at[pl.ds(base, stage)],
                            src_v.at[pl.ds(0, stage)])
            pltpu.sync_copy(idx2d.at[pl.ds(n_rows + base, stage)],
                            dst_v.at[pl.ds(0, stage)])

            bufs = ((rows0, gs0), (rows1, gs1))

            def gather(c, buf, gsem):
                return pltpu.async_copy(table.at[src_v.at[c]], buf, gsem)

            gather(0, rows0, gs0)
            gather(1, rows1, gs1)

            n2 = n_w // 2

            def iter_g(g, carry):
                for k, (buf, gsem) in enumerate(bufs):
                    c = 2 * g + k
                    pltpu.make_async_copy(table.at[src_v.at[c]],
                                          buf, gsem).wait()
                    pltpu.sync_copy(buf, acc.at[dst_v.at[c]], add=True)
                    if deg_acc is not None:
                        if k == 0:
                            @pl.when(g > 0)
                            def _():
                                pltpu.make_async_copy(
                                    ones_v, deg_acc.at[dst_v.at[c]],
                                    ds).wait()
                        else:
                            pltpu.make_async_copy(
                                ones_v, deg_acc.at[dst_v.at[c]], ds).wait()
                        pltpu.async_copy(ones_v, deg_acc.at[dst_v.at[c]],
                                         ds, add=True)

                    @pl.when(g < n2 - 1)
                    def _():
                        gather(c + 2, buf, gsem)
                return carry

            lax.fori_loop(0, n2, iter_g, 0)
            if deg_acc is not None:
                pltpu.make_async_copy(ones_v, deg_acc.at[dst_v.at[0]],
                                      ds).wait()

        def write_out(sh_acc, hbm_out):
            pltpu.sync_copy(sh_acc.at[pl.ds(stripe, STRIPE)],
                            hbm_out.at[pl.ds(cid * N + stripe, STRIPE)])

        run_pass(t_pos, pos_idx, POS_PW, POS_ROWS, POS_STAGE,
                 deg_p if with_deg else None)
        plsc.subcore_barrier()
        write_out(acc, out_p)
        if with_deg:
            write_out(deg_p, deg_p_out)
        pltpu.sync_copy(zf, acc.at[pl.ds(stripe, STRIPE)])
        plsc.subcore_barrier()
        run_pass(t_neg, neg_idx, NEG_PW, NEG_ROWS, NEG_STAGE,
                 deg_n if with_deg else None)
        plsc.subcore_barrier()
        write_out(acc, out_n)
        if with_deg:
            write_out(deg_n, deg_n_out)

    return pl.kernel(body, out_type=out_type, mesh=mesh,
                     scratch_types=scratch,
                     compiler_params=pltpu.CompilerParams(
                         use_tc_tiling_on_sc=False))


_seg32 = _seg_sum_kernel(H2, with_deg=True)
_seg64 = _seg_sum_kernel(2 * H2, with_deg=False)

ROW_BLK = 1056
GRID = N // ROW_BLK
NBLK = N // ROW_BLK


def _proj_body(x_ref, wp_ref, wn_ref, wr_ref, b_ref,
               tp_ref, tn_ref, lin_ref):
    x = x_ref[...]
    tp_ref[...] = jnp.dot(x, wp_ref[...], preferred_element_type=jnp.float32)
    tn_ref[...] = jnp.dot(x, wn_ref[...], preferred_element_type=jnp.float32)
    lin_ref[...] = (jnp.dot(x, wr_ref[...], preferred_element_type=jnp.float32)
                    + b_ref[...])


_proj = pl.pallas_call(
    _proj_body,
    grid=(GRID,),
    in_specs=[
        pl.BlockSpec((ROW_BLK, IN_CH), lambda i: (i, 0)),
        pl.BlockSpec((IN_CH, H2), lambda i: (0, 0)),
        pl.BlockSpec((IN_CH, H2), lambda i: (0, 0)),
        pl.BlockSpec((IN_CH, 2 * H2), lambda i: (0, 0)),
        pl.BlockSpec((1, 2 * H2), lambda i: (0, 0)),
    ],
    out_specs=[
        pl.BlockSpec((ROW_BLK, H2), lambda i: (i, 0)),
        pl.BlockSpec((ROW_BLK, H2), lambda i: (i, 0)),
        pl.BlockSpec((ROW_BLK, 2 * H2), lambda i: (i, 0)),
    ],
    out_shape=[
        jax.ShapeDtypeStruct((N, H2), jnp.float32),
        jax.ShapeDtypeStruct((N, H2), jnp.float32),
        jax.ShapeDtypeStruct((N, 2 * H2), jnp.float32),
    ],
)


def _pair_specs(feat_w):
    return [
        pl.BlockSpec((ROW_BLK, feat_w), lambda i: (i, 0)),
        pl.BlockSpec((ROW_BLK, feat_w), lambda i: (NBLK + i, 0)),
    ]


def _recip_deg(d0, d1):
    d = d0[:, 0:1] + d1[:, 0:1]
    return 1.0 / jnp.maximum(d, 1.0)


def _z_body(ap0_ref, ap1_ref, an0_ref, an1_ref, dp0_ref, dp1_ref,
            dn0_ref, dn1_ref, lin_ref, z_ref):
    sp = ap0_ref[...] + ap1_ref[...]
    sn = an0_ref[...] + an1_ref[...]
    rp = _recip_deg(dp0_ref[...], dp1_ref[...])
    rn = _recip_deg(dn0_ref[...], dn1_ref[...])
    agg = jnp.concatenate([sp * rp, sn * rn], axis=1)
    z_ref[...] = jnp.maximum(agg + lin_ref[...], 0.0)


_zker = pl.pallas_call(
    _z_body,
    grid=(GRID,),
    in_specs=(_pair_specs(H2) + _pair_specs(H2)
              + _pair_specs(16) + _pair_specs(16)
              + [pl.BlockSpec((ROW_BLK, 2 * H2), lambda i: (i, 0))]),
    out_specs=pl.BlockSpec((ROW_BLK, 2 * H2), lambda i: (i, 0)),
    out_shape=jax.ShapeDtypeStruct((N, 2 * H2), jnp.float32),
)


def _final_body(ap0_ref, ap1_ref, an0_ref, an1_ref, dp0_ref, dp1_ref,
                dn0_ref, dn1_ref, z_ref, m1_ref, m2_ref, m3_ref, b_ref,
                out_ref):
    rp = _recip_deg(dp0_ref[...], dp1_ref[...])
    rn = _recip_deg(dn0_ref[...], dn1_ref[...])
    mp = (ap0_ref[...] + ap1_ref[...]) * rp
    mn = (an0_ref[...] + an1_ref[...]) * rn
    acc = jnp.dot(mp, m1_ref[...], preferred_element_type=jnp.float32)
    acc += jnp.dot(mn, m2_ref[...], preferred_element_type=jnp.float32)
    acc += jnp.dot(z_ref[...], m3_ref[...], preferred_element_type=jnp.float32)
    out_ref[...] = jnp.maximum(acc + b_ref[...], 0.0)


_finalker = pl.pallas_call(
    _final_body,
    grid=(GRID,),
    in_specs=(_pair_specs(2 * H2) + _pair_specs(2 * H2)
              + _pair_specs(16) + _pair_specs(16)
              + [
                  pl.BlockSpec((ROW_BLK, 2 * H2), lambda i: (i, 0)),
                  pl.BlockSpec((2 * H2, 2 * H2), lambda i: (0, 0)),
                  pl.BlockSpec((2 * H2, 2 * H2), lambda i: (0, 0)),
                  pl.BlockSpec((2 * H2, 2 * H2), lambda i: (0, 0)),
                  pl.BlockSpec((1, 2 * H2), lambda i: (0, 0)),
              ]),
    out_specs=pl.BlockSpec((ROW_BLK, 2 * H2), lambda i: (i, 0)),
    out_shape=jax.ShapeDtypeStruct((N, 2 * H2), jnp.float32),
)


def kernel(x, pos_edge_index, neg_edge_index, x_param, W1_pl, W1_pr, b1_p,
           W1_nl, W1_nr, b1_n, W2_pl, W2_pr, b2_p, W2_nl, W2_nr, b2_n):
    del x

    pos_idx = pos_edge_index.reshape(2 * POS_ROWS, CHUNK)
    neg_idx = neg_edge_index.reshape(2 * NEG_ROWS, CHUNK)

    zf32 = jnp.zeros((STRIPE, H2), jnp.float32)
    zf64 = jnp.zeros((STRIPE, 2 * H2), jnp.float32)
    zd = jnp.zeros((STRIPE, 16), jnp.float32)
    ones_h = jnp.ones((CHUNK, 16), jnp.float32)

    wr1 = jnp.concatenate([W1_pr, W1_nr], axis=1)
    b1 = jnp.concatenate([b1_p, b1_n]).reshape(1, 2 * H2)

    t_pos, t_neg, lin = _proj(x_param, W1_pl, W1_nl, wr1, b1)
    accp, accn, degp, degn = _seg32(t_pos, t_neg, pos_idx, neg_idx,
                                    zf32, zd, ones_h)

    z = _zker(accp, accp, accn, accn, degp, degp, degn, degn, lin)

    accp2, accn2 = _seg64(z, z, pos_idx, neg_idx, zf64, zd, ones_h)

    zero = jnp.zeros((H2, H2), jnp.float32)
    A, B = W2_pl[:H2], W2_pl[H2:]
    C, D = W2_nl[:H2], W2_nl[H2:]
    m1 = jnp.concatenate([jnp.concatenate([A, zero], axis=1),
                          jnp.concatenate([zero, C], axis=1)], axis=0)
    m2 = jnp.concatenate([jnp.concatenate([zero, D], axis=1),
                          jnp.concatenate([B, zero], axis=1)], axis=0)
    m3 = jnp.concatenate([jnp.concatenate([W2_pr, zero], axis=1),
                          jnp.concatenate([zero, W2_nr], axis=1)], axis=0)
    b2 = jnp.concatenate([b2_p, b2_n]).reshape(1, 2 * H2)

    return _finalker(accp2, accp2, accn2, accn2, degp, degp, degn, degn,
                     z, m1, m2, m3, b2)


# speedup vs baseline: 30.5817x; 1.1744x over previous
import functools

import jax
import jax.numpy as jnp
from jax import lax
from jax.experimental import pallas as pl
from jax.experimental.pallas import tpu as pltpu
from jax.experimental.pallas import tpu_sc as plsc

N = 16896
IN_CH = 128
H2 = 32
E_POS = 540672
E_NEG = 270336

NC = 2
NS = 16
NW = NC * NS
CHUNK = 128

POS_ROWS = E_POS // CHUNK
NEG_ROWS = E_NEG // CHUNK
POS_PW = POS_ROWS // NW
NEG_PW = NEG_ROWS // NW
POS_STAGE = POS_PW + 4
NEG_STAGE = NEG_PW + 6
STRIPE = N // NS


def _seg_sum_kernel(feat_w, with_deg):
    out_type = jax.ShapeDtypeStruct((NC * N, 128), jnp.float32)
    scratch = [
        pltpu.VMEM((POS_STAGE, CHUNK), jnp.int32),
        pltpu.VMEM((POS_STAGE, CHUNK), jnp.int32),
        pltpu.VMEM((CHUNK, feat_w), jnp.float32),
        pltpu.VMEM((CHUNK, feat_w), jnp.float32),
        pltpu.VMEM_SHARED((N, feat_w), jnp.float32),
        pltpu.SemaphoreType.DMA,
        pltpu.SemaphoreType.DMA,
    ]
    if with_deg:
        scratch += [
            pltpu.VMEM((CHUNK, 16), jnp.float32),
            pltpu.VMEM_SHARED((N, 16), jnp.float32),
            pltpu.VMEM_SHARED((N, 16), jnp.float32),
            pltpu.SemaphoreType.DMA,
        ]

    mesh = plsc.VectorSubcoreMesh(core_axis_name="c", subcore_axis_name="s",
                                  num_cores=NC, num_subcores=NS)

    def body(t_pos, t_neg, pos_idx, neg_idx, zf, zd, ones_h,
             out_hbm, *rest):
        src_v, dst_v, rows0, rows1, acc, gs0, gs1 = rest[:7]
        if with_deg:
            ones_v, deg_p, deg_n, ds = rest[7:]

        cid = lax.axis_index("c")
        sid = lax.axis_index("s")
        wid = sid * NC + cid
        stripe = sid * STRIPE

        pltpu.sync_copy(zf, acc.at[pl.ds(stripe, STRIPE)])
        if with_deg:
            pltpu.sync_copy(zd, deg_p.at[pl.ds(stripe, STRIPE)])
            pltpu.sync_copy(zd, deg_n.at[pl.ds(stripe, STRIPE)])
            pltpu.sync_copy(ones_h, ones_v)
        plsc.subcore_barrier()

        def run_pass(table, idx2d, pw, n_rows, stage, deg_acc):
            base = (pw * wid) // 8 * 8
            n_w = (pw * wid + pw) // 8 * 8 - base
            pltpu.sync_copy(idx2d.at[pl.ds(base, stage)],
                            src_v.at[pl.ds(0, stage)])
            pltpu.sync_copy(idx2d.at[pl.ds(n_rows + base, stage)],
                            dst_v.at[pl.ds(0, stage)])

            bufs = ((rows0, gs0), (rows1, gs1))

            def gather(c, buf, gsem):
                return pltpu.async_copy(table.at[src_v.at[c]], buf, gsem)

            gather(0, rows0, gs0)
            gather(1, rows1, gs1)

            n2 = n_w // 2

            def iter_g(g, carry):
                for k, (buf, gsem) in enumerate(bufs):
                    c = 2 * g + k
                    pltpu.make_async_copy(table.at[src_v.at[c]],
                                          buf, gsem).wait()
                    pltpu.sync_copy(buf, acc.at[dst_v.at[c]], add=True)
                    if deg_acc is not None:
                        if k == 0:
                            @pl.when(g > 0)
                            def _():
                                pltpu.make_async_copy(
                                    ones_v, deg_acc.at[dst_v.at[c - 1]],
                                    ds).wait()
                        else:
                            pltpu.make_async_copy(
                                ones_v, deg_acc.at[dst_v.at[c - 1]],
                                ds).wait()
                        pltpu.async_copy(ones_v, deg_acc.at[dst_v.at[c]],
                                         ds, add=True)

                    @pl.when(g < n2 - 1)
                    def _():
                        gather(c + 2, buf, gsem)
                return carry

            lax.fori_loop(0, n2, iter_g, 0)
            if deg_acc is not None:
                pltpu.make_async_copy(ones_v, deg_acc.at[dst_v.at[n_w - 1]],
                                      ds).wait()

        def write_out(sh_acc, col, width):
            pltpu.sync_copy(
                sh_acc.at[pl.ds(stripe, STRIPE)],
                out_hbm.at[pl.ds(cid * N + stripe, STRIPE),
                           pl.ds(col, width)])

        run_pass(t_pos, pos_idx, POS_PW, POS_ROWS, POS_STAGE,
                 deg_p if with_deg else None)
        plsc.subcore_barrier()
        write_out(acc, 0, feat_w)
        if with_deg:
            write_out(deg_p, 2 * feat_w, 16)
        pltpu.sync_copy(zf, acc.at[pl.ds(stripe, STRIPE)])
        plsc.subcore_barrier()
        run_pass(t_neg, neg_idx, NEG_PW, NEG_ROWS, NEG_STAGE,
                 deg_n if with_deg else None)
        plsc.subcore_barrier()
        write_out(acc, feat_w, feat_w)
        if with_deg:
            write_out(deg_n, 2 * feat_w + 16, 16)

    return pl.kernel(body, out_type=out_type, mesh=mesh,
                     scratch_types=scratch,
                     compiler_params=pltpu.CompilerParams(
                         use_tc_tiling_on_sc=False))


_seg32 = _seg_sum_kernel(H2, with_deg=True)
_seg64 = _seg_sum_kernel(2 * H2, with_deg=False)

ROW_BLK = 1056
GRID = N // ROW_BLK
NBLK = N // ROW_BLK


def _proj_body(x_ref, wp_ref, wn_ref, wr_ref, b_ref,
               tp_ref, tn_ref, lin_ref):
    x = x_ref[...]
    tp_ref[...] = jnp.dot(x, wp_ref[...], preferred_element_type=jnp.float32)
    tn_ref[...] = jnp.dot(x, wn_ref[...], preferred_element_type=jnp.float32)
    lin_ref[...] = (jnp.dot(x, wr_ref[...], preferred_element_type=jnp.float32)
                    + b_ref[...])


_proj = pl.pallas_call(
    _proj_body,
    grid=(GRID,),
    in_specs=[
        pl.BlockSpec((ROW_BLK, IN_CH), lambda i: (i, 0)),
        pl.BlockSpec((IN_CH, H2), lambda i: (0, 0)),
        pl.BlockSpec((IN_CH, H2), lambda i: (0, 0)),
        pl.BlockSpec((IN_CH, 2 * H2), lambda i: (0, 0)),
        pl.BlockSpec((1, 2 * H2), lambda i: (0, 0)),
    ],
    out_specs=[
        pl.BlockSpec((ROW_BLK, H2), lambda i: (i, 0)),
        pl.BlockSpec((ROW_BLK, H2), lambda i: (i, 0)),
        pl.BlockSpec((ROW_BLK, 2 * H2), lambda i: (i, 0)),
    ],
    out_shape=[
        jax.ShapeDtypeStruct((N, H2), jnp.float32),
        jax.ShapeDtypeStruct((N, H2), jnp.float32),
        jax.ShapeDtypeStruct((N, 2 * H2), jnp.float32),
    ],
)


def _pair_specs():
    return [
        pl.BlockSpec((ROW_BLK, 128), lambda i: (i, 0)),
        pl.BlockSpec((ROW_BLK, 128), lambda i: (NBLK + i, 0)),
    ]


def _deg_recips(q0, q1):
    dp = q0[:, 64:65] + q1[:, 64:65]
    dn = q0[:, 80:81] + q1[:, 80:81]
    return (1.0 / jnp.maximum(dp, 1.0), 1.0 / jnp.maximum(dn, 1.0))


def _z_body(q0_ref, q1_ref, lin_ref, z_ref):
    q0 = q0_ref[...]
    q1 = q1_ref[...]
    sp = q0[:, 0:H2] + q1[:, 0:H2]
    sn = q0[:, H2:2 * H2] + q1[:, H2:2 * H2]
    rp, rn = _deg_recips(q0, q1)
    agg = jnp.concatenate([sp * rp, sn * rn], axis=1)
    z_ref[...] = jnp.maximum(agg + lin_ref[...], 0.0)


_zker = pl.pallas_call(
    _z_body,
    grid=(GRID,),
    in_specs=(_pair_specs()
              + [pl.BlockSpec((ROW_BLK, 2 * H2), lambda i: (i, 0))]),
    out_specs=pl.BlockSpec((ROW_BLK, 2 * H2), lambda i: (i, 0)),
    out_shape=jax.ShapeDtypeStruct((N, 2 * H2), jnp.float32),
)


def _final_body(p0_ref, p1_ref, q0_ref, q1_ref, z_ref, m1_ref, m2_ref,
                m3_ref, b_ref, out_ref):
    p0 = p0_ref[...]
    p1 = p1_ref[...]
    rp, rn = _deg_recips(q0_ref[...], q1_ref[...])
    mp = (p0[:, :2 * H2] + p1[:, :2 * H2]) * rp
    mn = (p0[:, 2 * H2:] + p1[:, 2 * H2:]) * rn
    acc = jnp.dot(mp, m1_ref[...], preferred_element_type=jnp.float32)
    acc += jnp.dot(mn, m2_ref[...], preferred_element_type=jnp.float32)
    acc += jnp.dot(z_ref[...], m3_ref[...], preferred_element_type=jnp.float32)
    out_ref[...] = jnp.maximum(acc + b_ref[...], 0.0)


_finalker = pl.pallas_call(
    _final_body,
    grid=(GRID,),
    in_specs=(_pair_specs() + _pair_specs()
              + [
                  pl.BlockSpec((ROW_BLK, 2 * H2), lambda i: (i, 0)),
                  pl.BlockSpec((2 * H2, 2 * H2), lambda i: (0, 0)),
                  pl.BlockSpec((2 * H2, 2 * H2), lambda i: (0, 0)),
                  pl.BlockSpec((2 * H2, 2 * H2), lambda i: (0, 0)),
                  pl.BlockSpec((1, 2 * H2), lambda i: (0, 0)),
              ]),
    out_specs=pl.BlockSpec((ROW_BLK, 2 * H2), lambda i: (i, 0)),
    out_shape=jax.ShapeDtypeStruct((N, 2 * H2), jnp.float32),
)


def kernel(x, pos_edge_index, neg_edge_index, x_param, W1_pl, W1_pr, b1_p,
           W1_nl, W1_nr, b1_n, W2_pl, W2_pr, b2_p, W2_nl, W2_nr, b2_n):
    del x

    pos_idx = pos_edge_index.reshape(2 * POS_ROWS, CHUNK)
    neg_idx = neg_edge_index.reshape(2 * NEG_ROWS, CHUNK)

    zf32 = jnp.zeros((STRIPE, H2), jnp.float32)
    zf64 = jnp.zeros((STRIPE, 2 * H2), jnp.float32)
    zd = jnp.zeros((STRIPE, 16), jnp.float32)
    ones_h = jnp.ones((CHUNK, 16), jnp.float32)

    wr1 = jnp.concatenate([W1_pr, W1_nr], axis=1)
    b1 = jnp.concatenate([b1_p, b1_n]).reshape(1, 2 * H2)

    t_pos, t_neg, lin = _proj(x_param, W1_pl, W1_nl, wr1, b1)
    q = _seg32(t_pos, t_neg, pos_idx, neg_idx, zf32, zd, ones_h)

    z = _zker(q, q, lin)

    p = _seg64(z, z, pos_idx, neg_idx, zf64, zd, ones_h)

    zero = jnp.zeros((H2, H2), jnp.float32)
    A, B = W2_pl[:H2], W2_pl[H2:]
    C, D = W2_nl[:H2], W2_nl[H2:]
    m1 = jnp.concatenate([jnp.concatenate([A, zero], axis=1),
                          jnp.concatenate([zero, C], axis=1)], axis=0)
    m2 = jnp.concatenate([jnp.concatenate([zero, D], axis=1),
                          jnp.concatenate([B, zero], axis=1)], axis=0)
    m3 = jnp.concatenate([jnp.concatenate([W2_pr, zero], axis=1),
                          jnp.concatenate([zero, W2_nr], axis=1)], axis=0)
    b2 = jnp.concatenate([b2_p, b2_n]).reshape(1, 2 * H2)

    return _finalker(p, p, q, q, z, m1, m2, m3, b2)
